# baseline (speedup 1.0000x reference)
import functools

import jax
import jax.numpy as jnp
import numpy as np
from jax import lax
from jax.experimental import pallas as pl
from jax.experimental.pallas import tpu as pltpu
from jax.experimental.pallas import tpu_sc as plsc

N = 10000
E = 320000
R = 1001
D = 128

NUM_TILES = 32
C = 64
CHUNKS = 4 * (-(-E // (4 * C * NUM_TILES)))
CH_ALLOC = CHUNKS + 2
EPT = CHUNKS * C
E_PAD = EPT * NUM_TILES
N_PAD = 10240
R_PAD = 1008
ROWS_PER_TILE = N_PAD // 16

_mesh = plsc.VectorSubcoreMesh(core_axis_name="c", subcore_axis_name="s")


def _deg_body(dst_hbm, deg_out,
              dst_v0, dst_v1, sdst_v0, sdst_v1, ones_v, zero_v, deg_sh,
              isem0, isem1, ssem0, ssem1):
    c = lax.axis_index("c")
    s = lax.axis_index("s")
    wid = s * 2 + c

    zeros16 = jnp.zeros((16,), jnp.float32)
    ones16 = jnp.ones((16,), jnp.float32)

    def fill(i, _):
        for k in range(D // 16):
            ones_v[i, pl.ds(k * 16, 16)] = ones16
            zero_v[i, pl.ds(k * 16, 16)] = zeros16
        return 0
    lax.fori_loop(0, C, fill, 0)

    row0 = s * ROWS_PER_TILE
    for j in range(ROWS_PER_TILE // C):
        pltpu.sync_copy(zero_v, deg_sh.at[pl.ds(row0 + j * C, C)])

    plsc.subcore_barrier()

    base_edge = wid * CH_ALLOC * C
    bufs = ((dst_v0, sdst_v0, isem0, ssem0), (dst_v1, sdst_v1, isem1, ssem1))

    def start_idx(ci, b):
        dstv, _, isem, _ = bufs[b]
        pltpu.async_copy(dst_hbm.at[pl.ds(base_edge + ci * C, C)], dstv, isem)

    def wait_idx(b):
        dstv, _, isem, _ = bufs[b]
        pltpu.make_async_copy(dst_hbm.at[pl.ds(0, C)], dstv, isem).wait()

    def start_scatter(b):
        _, sdstv, _, ssem = bufs[b]
        pltpu.async_copy(ones_v, deg_sh.at[sdstv], ssem, add=True)

    def wait_scatter(b):
        _, _, _, ssem = bufs[b]
        pltpu.make_async_copy(deg_out.at[0, pl.ds(0, C)], ones_v, ssem).wait()

    def iter_body(ci, b, first):
        dstv, sdstv, _, _ = bufs[b]
        wait_idx(b)
        if not first:
            wait_scatter(1 - b)
        for k in range(C // 16):
            sdstv[pl.ds(k * 16, 16)] = dstv[pl.ds(k * 16, 16)]
        start_scatter(b)
        start_idx(ci + 2, b)

    start_idx(0, 0)
    start_idx(1, 1)
    iter_body(0, 0, first=True)
    iter_body(1, 1, first=False)

    def pair_body(g, _):
        for b in (0, 1):
            iter_body(2 * g + b, b, first=False)
        return 0

    lax.fori_loop(1, CHUNKS // 2, pair_body, 0)

    wait_idx(0)
    wait_idx(1)
    wait_scatter(1)

    plsc.subcore_barrier()

    pltpu.sync_copy(deg_sh.at[pl.ds(row0, ROWS_PER_TILE)],
                    deg_out.at[c, pl.ds(row0, ROWS_PER_TILE)])


_deg_pass = functools.partial(
    pl.kernel,
    out_type=jax.ShapeDtypeStruct((2, N_PAD, D), jnp.float32),
    mesh=_mesh,
    scratch_types=[
        pltpu.VMEM((C,), jnp.int32),
        pltpu.VMEM((C,), jnp.int32),
        pltpu.VMEM((C,), jnp.int32),
        pltpu.VMEM((C,), jnp.int32),
        pltpu.VMEM((C, D), jnp.float32),
        pltpu.VMEM((C, D), jnp.float32),
        pltpu.VMEM_SHARED((N_PAD, D), jnp.float32),
        pltpu.SemaphoreType.DMA,
        pltpu.SemaphoreType.DMA,
        pltpu.SemaphoreType.DMA,
        pltpu.SemaphoreType.DMA,
    ],
)(_deg_body)


def _edge_pass_body(x_hbm, p_hbm, st_hbm, dst_hbm,
                    agg_out,
                    st_v0, st_v1, dst_v0, dst_v1, sdst_v0, sdst_v1,
                    h_v0, h_v1, p_v0, p_v1,
                    agg_sh,
                    isem0, isem1, hsem0, hsem1, psem0, psem1, ssem0, ssem1):
    c = lax.axis_index("c")
    s = lax.axis_index("s")
    wid = s * 2 + c

    zeros16 = jnp.zeros((16,), jnp.float32)

    def zrow(i, _):
        for k in range(D // 16):
            h_v0[i, pl.ds(k * 16, 16)] = zeros16
        return 0
    lax.fori_loop(0, C, zrow, 0)

    row0 = s * ROWS_PER_TILE
    for j in range(ROWS_PER_TILE // C):
        pltpu.sync_copy(h_v0, agg_sh.at[pl.ds(row0 + j * C, C)])

    plsc.subcore_barrier()

    base_st = wid * CH_ALLOC * 2 * C
    base_dst = wid * CH_ALLOC * C
    bufs = ((st_v0, dst_v0, sdst_v0, h_v0, p_v0, isem0, hsem0, psem0, ssem0),
            (st_v1, dst_v1, sdst_v1, h_v1, p_v1, isem1, hsem1, psem1, ssem1))

    def start_idx(ci, b):
        stv, dstv, _, _, _, isem, _, _, _ = bufs[b]
        pltpu.async_copy(st_hbm.at[pl.ds(base_st + ci * 2 * C, 2 * C)],
                         stv, isem)
        pltpu.async_copy(dst_hbm.at[pl.ds(base_dst + ci * C, C)],
                         dstv, isem)

    def wait_idx(b):
        stv, dstv, _, _, _, isem, _, _, _ = bufs[b]
        pltpu.make_async_copy(st_hbm.at[pl.ds(0, 2 * C)], stv, isem).wait()
        pltpu.make_async_copy(dst_hbm.at[pl.ds(0, C)], dstv, isem).wait()

    def start_gather(b):
        stv, _, _, hv, pv, _, hsem, psem, _ = bufs[b]
        pltpu.async_copy(x_hbm.at[stv.at[pl.ds(0, C)]], hv, hsem)
        pltpu.async_copy(p_hbm.at[stv.at[pl.ds(C, C)]], pv, psem)

    def wait_gather(b):
        _, _, _, hv, pv, _, hsem, psem, _ = bufs[b]
        pltpu.make_async_copy(x_hbm.at[pl.ds(0, C)], hv, hsem).wait()
        pltpu.make_async_copy(x_hbm.at[pl.ds(0, C)], pv, psem).wait()

    def start_scatter(b):
        _, _, sdstv, hv, _, _, _, _, ssem = bufs[b]
        pltpu.async_copy(hv, agg_sh.at[sdstv], ssem, add=True)

    def wait_scatter(b):
        _, _, _, hv, _, _, _, _, ssem = bufs[b]
        pltpu.make_async_copy(x_hbm.at[pl.ds(0, C)], hv, ssem).wait()

    lanes = lax.iota(jnp.int32, 16)
    _gdn = lax.GatherDimensionNumbers(
        offset_dims=(), collapsed_slice_dims=(0,), start_index_map=(0,))

    def _shuf(v, idx):
        return lax.gather(v, idx[:, None], _gdn, (1,),
                          mode=lax.GatherScatterMode.PROMISE_IN_BOUNDS)

    def compute_and_scatter(b):
        _, dstv, sdstv, hv, pv, _, _, _, _ = bufs[b]

        def edge_body(e, _):
            acc = zeros16
            for k in range(D // 16):
                acc = acc + hv[e, pl.ds(k * 16, 16)] * pv[e, pl.ds(k * 16, 16)]
            for sh in (8, 4, 2, 1):
                acc = acc + _shuf(acc, lanes ^ sh)
            for k in range(D // 16):
                hv[e, pl.ds(k * 16, 16)] = (
                    hv[e, pl.ds(k * 16, 16)] - acc * pv[e, pl.ds(k * 16, 16)]
                )
            return 0
        lax.fori_loop(0, C, edge_body, 0)

        for k in range(C // 16):
            sdstv[pl.ds(k * 16, 16)] = dstv[pl.ds(k * 16, 16)]
        start_scatter(b)

    def iter_body(ci, b, first):
        with jax.named_scope("ew_idx"):
            wait_idx(1 - b)
        if not first:
            with jax.named_scope("ew_scat"):
                wait_scatter(1 - b)
        start_gather(1 - b)
        with jax.named_scope("ew_gath"):
            wait_gather(b)
        with jax.named_scope("ew_comp"):
            compute_and_scatter(b)
        start_idx(ci + 2, b)

    start_idx(0, 0)
    wait_idx(0)
    start_gather(0)
    start_idx(1, 1)
    iter_body(0, 0, first=True)
    iter_body(1, 1, first=False)

    def pair_body(g, _):
        for b in (0, 1):
            iter_body(2 * g + b, b, first=False)
        return 0

    lax.fori_loop(1, CHUNKS // 2, pair_body, 0)

    wait_gather(0)
    wait_idx(1)
    wait_scatter(1)

    plsc.subcore_barrier()

    pltpu.sync_copy(agg_sh.at[pl.ds(row0, ROWS_PER_TILE)],
                    agg_out.at[c, pl.ds(row0, ROWS_PER_TILE)])


_edge_pass = functools.partial(
    pl.kernel,
    out_type=jax.ShapeDtypeStruct((2, N_PAD, D), jnp.float32),
    mesh=_mesh,
    scratch_types=[
        pltpu.VMEM((2 * C,), jnp.int32),
        pltpu.VMEM((2 * C,), jnp.int32),
        pltpu.VMEM((C,), jnp.int32),
        pltpu.VMEM((C,), jnp.int32),
        pltpu.VMEM((C,), jnp.int32),
        pltpu.VMEM((C,), jnp.int32),
        pltpu.VMEM((C, D), jnp.float32),
        pltpu.VMEM((C, D), jnp.float32),
        pltpu.VMEM((C, D), jnp.float32),
        pltpu.VMEM((C, D), jnp.float32),
        pltpu.VMEM_SHARED((N_PAD, D), jnp.float32),
        pltpu.SemaphoreType.DMA,
        pltpu.SemaphoreType.DMA,
        pltpu.SemaphoreType.DMA,
        pltpu.SemaphoreType.DMA,
        pltpu.SemaphoreType.DMA,
        pltpu.SemaphoreType.DMA,
        pltpu.SemaphoreType.DMA,
        pltpu.SemaphoreType.DMA,
    ],
)(_edge_pass_body)


def _prep_body(r_ref, o_ref):
    r = r_ref[...]
    norm = jnp.sqrt(jnp.sum(r * r, axis=1, keepdims=True))
    o_ref[...] = r * (np.float32(np.sqrt(2.0)) / (norm + 1e-8))


_prep = pl.pallas_call(
    _prep_body,
    out_shape=jax.ShapeDtypeStruct((R_PAD, D), jnp.float32),
)


def _layer_body(a_ref, d_ref, w_ref, o_ref):
    deg = jnp.maximum(d_ref[0, :, 0:1] + d_ref[1, :, 0:1], 1.0)
    x = (a_ref[0] + a_ref[1]) / deg
    o_ref[...] = jnp.maximum(
        jnp.dot(x, w_ref[...], preferred_element_type=jnp.float32), 0.0)


_layer = pl.pallas_call(
    _layer_body,
    out_shape=jax.ShapeDtypeStruct((N_PAD, D), jnp.float32),
)


def _final_body(a_ref, d_ref, w2_ref, l1w_ref, l1b_ref, g_ref, b_ref,
                l2w_ref, l2b_ref, o_ref):
    deg = jnp.maximum(d_ref[0, :, 0:1] + d_ref[1, :, 0:1], 1.0)
    x = (a_ref[0] + a_ref[1]) / deg
    x = jnp.maximum(
        jnp.dot(x, w2_ref[...], preferred_element_type=jnp.float32), 0.0)
    h = jnp.dot(x, l1w_ref[...], preferred_element_type=jnp.float32) + l1b_ref[...]
    mask = (lax.broadcasted_iota(jnp.int32, (N_PAD, 1), 0) < N).astype(jnp.float32)
    cnt = np.float32(N)
    mean = jnp.sum(h * mask, axis=0, keepdims=True) / cnt
    var = jnp.sum((h - mean) ** 2 * mask, axis=0, keepdims=True) / cnt
    h = (h - mean) / jnp.sqrt(var + 1e-5) * g_ref[...] + b_ref[...]
    h = jnp.maximum(h, 0.0)
    o_ref[...] = jnp.dot(h, l2w_ref[...], preferred_element_type=jnp.float32) + l2b_ref[...]


_final = pl.pallas_call(
    _final_body,
    out_shape=jax.ShapeDtypeStruct((N_PAD, D), jnp.float32),
)


def kernel(edge_index, edge_type, initial_features, relation_embeddings,
           W1, W2, lin1_w, lin1_b, bn_gamma, bn_beta, lin2_w, lin2_b):
    pad = E_PAD - E
    src = jnp.concatenate(
        [edge_index[0].astype(jnp.int32), jnp.zeros((pad,), jnp.int32)])
    dst = jnp.concatenate(
        [edge_index[1].astype(jnp.int32), jnp.full((pad,), N, jnp.int32)])
    typ = jnp.concatenate(
        [edge_type.astype(jnp.int32), jnp.zeros((pad,), jnp.int32)])

    src_r = jnp.pad(src.reshape(NUM_TILES, CHUNKS, C), ((0, 0), (0, 2), (0, 0)))
    typ_r = jnp.pad(typ.reshape(NUM_TILES, CHUNKS, C), ((0, 0), (0, 2), (0, 0)))
    st = jnp.stack([src_r, typ_r], axis=2).reshape(-1)
    dstc = jnp.pad(dst.reshape(NUM_TILES, CHUNKS, C), ((0, 0), (0, 2), (0, 0)),
                   constant_values=N).reshape(-1)

    x0 = jnp.pad(initial_features, ((0, N_PAD - N), (0, 0)))
    relp = jnp.pad(relation_embeddings, ((0, R_PAD - R), (0, 0)))

    p = _prep(relp)
    deg = _deg_pass(dstc)

    agg1 = _edge_pass(x0, p, st, dstc)
    x1 = _layer(agg1, deg, W1)
    agg2 = _edge_pass(x1, p, st, dstc)
    out = _final(agg2, deg, W2, lin1_w, lin1_b.reshape(1, D),
                 bn_gamma.reshape(1, D), bn_beta.reshape(1, D),
                 lin2_w, lin2_b.reshape(1, D))
    return out[:N]


# speedup vs baseline: 1.0511x; 1.0511x over previous
import functools

import jax
import jax.numpy as jnp
import numpy as np
from jax import lax
from jax.experimental import pallas as pl
from jax.experimental.pallas import tpu as pltpu
from jax.experimental.pallas import tpu_sc as plsc

N = 10000
E = 320000
R = 1001
D = 128

NUM_TILES = 32
C = 64
CHUNKS = 4 * (-(-E // (4 * C * NUM_TILES)))
CH_ALLOC = CHUNKS + 2
EPT = CHUNKS * C
E_PAD = EPT * NUM_TILES
N_PAD = 10240
R_PAD = 1008
ROWS_PER_TILE = N_PAD // 16

_mesh = plsc.VectorSubcoreMesh(core_axis_name="c", subcore_axis_name="s")


def _deg_body(dst_hbm, deg_out,
              dst_v0, dst_v1, sdst_v0, sdst_v1, ones_v, zero_v, deg_sh,
              isem0, isem1, ssem0, ssem1):
    c = lax.axis_index("c")
    s = lax.axis_index("s")
    wid = s * 2 + c

    zeros16 = jnp.zeros((16,), jnp.float32)
    ones16 = jnp.ones((16,), jnp.float32)

    def fill(i, _):
        for k in range(D // 16):
            ones_v[i, pl.ds(k * 16, 16)] = ones16
            zero_v[i, pl.ds(k * 16, 16)] = zeros16
        return 0
    lax.fori_loop(0, C, fill, 0)

    row0 = s * ROWS_PER_TILE
    for j in range(ROWS_PER_TILE // C):
        pltpu.sync_copy(zero_v, deg_sh.at[pl.ds(row0 + j * C, C)])

    plsc.subcore_barrier()

    base_edge = wid * CH_ALLOC * C
    bufs = ((dst_v0, sdst_v0, isem0, ssem0), (dst_v1, sdst_v1, isem1, ssem1))

    def start_idx(ci, b):
        dstv, _, isem, _ = bufs[b]
        pltpu.async_copy(dst_hbm.at[pl.ds(base_edge + ci * C, C)], dstv, isem)

    def wait_idx(b):
        dstv, _, isem, _ = bufs[b]
        pltpu.make_async_copy(dst_hbm.at[pl.ds(0, C)], dstv, isem).wait()

    def start_scatter(b):
        _, sdstv, _, ssem = bufs[b]
        pltpu.async_copy(ones_v, deg_sh.at[sdstv], ssem, add=True)

    def wait_scatter(b):
        _, _, _, ssem = bufs[b]
        pltpu.make_async_copy(deg_out.at[0, pl.ds(0, C)], ones_v, ssem).wait()

    def iter_body(ci, b, first):
        dstv, sdstv, _, _ = bufs[b]
        wait_idx(b)
        if not first:
            wait_scatter(1 - b)
        for k in range(C // 16):
            sdstv[pl.ds(k * 16, 16)] = dstv[pl.ds(k * 16, 16)]
        start_scatter(b)
        start_idx(ci + 2, b)

    start_idx(0, 0)
    start_idx(1, 1)
    iter_body(0, 0, first=True)
    iter_body(1, 1, first=False)

    def pair_body(g, _):
        for b in (0, 1):
            iter_body(2 * g + b, b, first=False)
        return 0

    lax.fori_loop(1, CHUNKS // 2, pair_body, 0)

    wait_idx(0)
    wait_idx(1)
    wait_scatter(1)

    plsc.subcore_barrier()

    pltpu.sync_copy(deg_sh.at[pl.ds(row0, ROWS_PER_TILE)],
                    deg_out.at[c, pl.ds(row0, ROWS_PER_TILE)])


_deg_pass = functools.partial(
    pl.kernel,
    out_type=jax.ShapeDtypeStruct((2, N_PAD, D), jnp.float32),
    mesh=_mesh,
    scratch_types=[
        pltpu.VMEM((C,), jnp.int32),
        pltpu.VMEM((C,), jnp.int32),
        pltpu.VMEM((C,), jnp.int32),
        pltpu.VMEM((C,), jnp.int32),
        pltpu.VMEM((C, D), jnp.float32),
        pltpu.VMEM((C, D), jnp.float32),
        pltpu.VMEM_SHARED((N_PAD, D), jnp.float32),
        pltpu.SemaphoreType.DMA,
        pltpu.SemaphoreType.DMA,
        pltpu.SemaphoreType.DMA,
        pltpu.SemaphoreType.DMA,
    ],
)(_deg_body)


def _edge_pass_body(x_hbm, p_hbm, st_hbm,
                    agg_out,
                    st_v0, st_v1, sdst_v0, sdst_v1,
                    h_v0, h_v1, p_v0, p_v1,
                    agg_sh, p_sh,
                    isem0, isem1, hsem0, hsem1, psem0, psem1, ssem0, ssem1):
    c = lax.axis_index("c")
    s = lax.axis_index("s")
    wid = s * 2 + c

    zeros16 = jnp.zeros((16,), jnp.float32)

    def zrow(i, _):
        for k in range(D // 16):
            h_v0[i, pl.ds(k * 16, 16)] = zeros16
        return 0
    lax.fori_loop(0, C, zrow, 0)

    @pl.when(s == 0)
    def _():
        pltpu.sync_copy(p_hbm, p_sh)

    row0 = s * ROWS_PER_TILE
    for j in range(ROWS_PER_TILE // C):
        pltpu.sync_copy(h_v0, agg_sh.at[pl.ds(row0 + j * C, C)])

    plsc.subcore_barrier()

    base_st = wid * CH_ALLOC * 3 * C
    bufs = ((st_v0, sdst_v0, h_v0, p_v0, isem0, hsem0, psem0, ssem0),
            (st_v1, sdst_v1, h_v1, p_v1, isem1, hsem1, psem1, ssem1))

    def start_idx(ci, b):
        stv, _, _, _, isem, _, _, _ = bufs[b]
        pltpu.async_copy(st_hbm.at[pl.ds(base_st + ci * 3 * C, 3 * C)],
                         stv, isem)

    def wait_idx(b):
        stv, _, _, _, isem, _, _, _ = bufs[b]
        pltpu.make_async_copy(st_hbm.at[pl.ds(0, 3 * C)], stv, isem).wait()

    def start_gather(b):
        stv, _, hv, pv, _, hsem, psem, _ = bufs[b]
        pltpu.async_copy(x_hbm.at[stv.at[pl.ds(0, C)]], hv, hsem)
        pltpu.async_copy(p_sh.at[stv.at[pl.ds(C, C)]], pv, psem)

    def wait_gather(b):
        _, _, hv, pv, _, hsem, psem, _ = bufs[b]
        pltpu.make_async_copy(x_hbm.at[pl.ds(0, C)], hv, hsem).wait()
        pltpu.make_async_copy(x_hbm.at[pl.ds(0, C)], pv, psem).wait()

    def start_scatter(b):
        _, sdstv, hv, _, _, _, _, ssem = bufs[b]
        pltpu.async_copy(hv, agg_sh.at[sdstv], ssem, add=True)

    def wait_scatter(b):
        _, _, hv, _, _, _, _, ssem = bufs[b]
        pltpu.make_async_copy(x_hbm.at[pl.ds(0, C)], hv, ssem).wait()

    lanes = lax.iota(jnp.int32, 16)
    _gdn = lax.GatherDimensionNumbers(
        offset_dims=(), collapsed_slice_dims=(0,), start_index_map=(0,))

    def _shuf(v, idx):
        return lax.gather(v, idx[:, None], _gdn, (1,),
                          mode=lax.GatherScatterMode.PROMISE_IN_BOUNDS)

    def compute_and_scatter(b):
        stv, sdstv, hv, pv, _, _, _, _ = bufs[b]

        def edge_body(e, _):
            acc = zeros16
            for k in range(D // 16):
                acc = acc + hv[e, pl.ds(k * 16, 16)] * pv[e, pl.ds(k * 16, 16)]
            for sh in (8, 4, 2, 1):
                acc = acc + _shuf(acc, lanes ^ sh)
            for k in range(D // 16):
                hv[e, pl.ds(k * 16, 16)] = (
                    hv[e, pl.ds(k * 16, 16)] - acc * pv[e, pl.ds(k * 16, 16)]
                )
            return 0
        lax.fori_loop(0, C, edge_body, 0)

        for k in range(C // 16):
            sdstv[pl.ds(k * 16, 16)] = stv[pl.ds(2 * C + k * 16, 16)]
        start_scatter(b)

    def iter_body(ci, b, first):
        wait_idx(1 - b)
        if not first:
            wait_scatter(1 - b)
        start_gather(1 - b)
        wait_gather(b)
        compute_and_scatter(b)
        start_idx(ci + 2, b)

    start_idx(0, 0)
    wait_idx(0)
    start_gather(0)
    start_idx(1, 1)
    iter_body(0, 0, first=True)
    iter_body(1, 1, first=False)

    def pair_body(g, _):
        for b in (0, 1):
            iter_body(2 * g + b, b, first=False)
        return 0

    lax.fori_loop(1, CHUNKS // 2, pair_body, 0)

    wait_gather(0)
    wait_idx(1)
    wait_scatter(1)

    plsc.subcore_barrier()

    pltpu.sync_copy(agg_sh.at[pl.ds(row0, ROWS_PER_TILE)],
                    agg_out.at[c, pl.ds(row0, ROWS_PER_TILE)])


_edge_pass = functools.partial(
    pl.kernel,
    out_type=jax.ShapeDtypeStruct((2, N_PAD, D), jnp.float32),
    mesh=_mesh,
    scratch_types=[
        pltpu.VMEM((3 * C,), jnp.int32),
        pltpu.VMEM((3 * C,), jnp.int32),
        pltpu.VMEM((C,), jnp.int32),
        pltpu.VMEM((C,), jnp.int32),
        pltpu.VMEM((C, D), jnp.float32),
        pltpu.VMEM((C, D), jnp.float32),
        pltpu.VMEM((C, D), jnp.float32),
        pltpu.VMEM((C, D), jnp.float32),
        pltpu.VMEM_SHARED((N_PAD, D), jnp.float32),
        pltpu.VMEM_SHARED((R_PAD, D), jnp.float32),
        pltpu.SemaphoreType.DMA,
        pltpu.SemaphoreType.DMA,
        pltpu.SemaphoreType.DMA,
        pltpu.SemaphoreType.DMA,
        pltpu.SemaphoreType.DMA,
        pltpu.SemaphoreType.DMA,
        pltpu.SemaphoreType.DMA,
        pltpu.SemaphoreType.DMA,
    ],
)(_edge_pass_body)


def _prep_body(r_ref, o_ref):
    r = r_ref[...]
    norm = jnp.sqrt(jnp.sum(r * r, axis=1, keepdims=True))
    o_ref[...] = r * (np.float32(np.sqrt(2.0)) / (norm + 1e-8))


_prep = pl.pallas_call(
    _prep_body,
    out_shape=jax.ShapeDtypeStruct((R_PAD, D), jnp.float32),
)


def _layer_body(a_ref, d_ref, w_ref, o_ref):
    deg = jnp.maximum(d_ref[0, :, 0:1] + d_ref[1, :, 0:1], 1.0)
    x = (a_ref[0] + a_ref[1]) / deg
    o_ref[...] = jnp.maximum(
        jnp.dot(x, w_ref[...], preferred_element_type=jnp.float32), 0.0)


_layer = pl.pallas_call(
    _layer_body,
    out_shape=jax.ShapeDtypeStruct((N_PAD, D), jnp.float32),
)


def _final_body(a_ref, d_ref, w2_ref, l1w_ref, l1b_ref, g_ref, b_ref,
                l2w_ref, l2b_ref, o_ref):
    deg = jnp.maximum(d_ref[0, :, 0:1] + d_ref[1, :, 0:1], 1.0)
    x = (a_ref[0] + a_ref[1]) / deg
    x = jnp.maximum(
        jnp.dot(x, w2_ref[...], preferred_element_type=jnp.float32), 0.0)
    h = jnp.dot(x, l1w_ref[...], preferred_element_type=jnp.float32) + l1b_ref[...]
    mask = (lax.broadcasted_iota(jnp.int32, (N_PAD, 1), 0) < N).astype(jnp.float32)
    cnt = np.float32(N)
    mean = jnp.sum(h * mask, axis=0, keepdims=True) / cnt
    var = jnp.sum((h - mean) ** 2 * mask, axis=0, keepdims=True) / cnt
    h = (h - mean) / jnp.sqrt(var + 1e-5) * g_ref[...] + b_ref[...]
    h = jnp.maximum(h, 0.0)
    o_ref[...] = jnp.dot(h, l2w_ref[...], preferred_element_type=jnp.float32) + l2b_ref[...]


_final = pl.pallas_call(
    _final_body,
    out_shape=jax.ShapeDtypeStruct((N_PAD, D), jnp.float32),
)


def kernel(edge_index, edge_type, initial_features, relation_embeddings,
           W1, W2, lin1_w, lin1_b, bn_gamma, bn_beta, lin2_w, lin2_b):
    pad = E_PAD - E
    src = jnp.concatenate(
        [edge_index[0].astype(jnp.int32), jnp.zeros((pad,), jnp.int32)])
    dst = jnp.concatenate(
        [edge_index[1].astype(jnp.int32), jnp.full((pad,), N, jnp.int32)])
    typ = jnp.concatenate(
        [edge_type.astype(jnp.int32), jnp.zeros((pad,), jnp.int32)])

    src_r = jnp.pad(src.reshape(NUM_TILES, CHUNKS, C), ((0, 0), (0, 2), (0, 0)))
    typ_r = jnp.pad(typ.reshape(NUM_TILES, CHUNKS, C), ((0, 0), (0, 2), (0, 0)))
    dst_r = jnp.pad(dst.reshape(NUM_TILES, CHUNKS, C), ((0, 0), (0, 2), (0, 0)),
                    constant_values=N)
    st = jnp.stack([src_r, typ_r, dst_r], axis=2).reshape(-1)
    dstc = dst_r.reshape(-1)

    x0 = jnp.pad(initial_features, ((0, N_PAD - N), (0, 0)))
    relp = jnp.pad(relation_embeddings, ((0, R_PAD - R), (0, 0)))

    p = _prep(relp)
    deg = _deg_pass(dstc)

    agg1 = _edge_pass(x0, p, st)
    x1 = _layer(agg1, deg, W1)
    agg2 = _edge_pass(x1, p, st)
    out = _final(agg2, deg, W2, lin1_w, lin1_b.reshape(1, D),
                 bn_gamma.reshape(1, D), bn_beta.reshape(1, D),
                 lin2_w, lin2_b.reshape(1, D))
    return out[:N]


# speedup vs baseline: 1.0559x; 1.0046x over previous
import functools

import jax
import jax.numpy as jnp
import numpy as np
from jax import lax
from jax.experimental import pallas as pl
from jax.experimental.pallas import tpu as pltpu
from jax.experimental.pallas import tpu_sc as plsc

N = 10000
E = 320000
R = 1001
D = 128

NUM_TILES = 32
C = 64
CHUNKS = 4 * (-(-E // (4 * C * NUM_TILES)))
CH_ALLOC = CHUNKS + 2
EPT = CHUNKS * C
E_PAD = EPT * NUM_TILES
N_PAD = 10240
R_PAD = 1008
ROWS_PER_TILE = N_PAD // 16

_mesh = plsc.VectorSubcoreMesh(core_axis_name="c", subcore_axis_name="s")


def _deg_body(dst_hbm, deg_out,
              dst_v0, dst_v1, sdst_v0, sdst_v1, ones_v, zero_v, deg_sh,
              isem0, isem1, ssem0, ssem1):
    c = lax.axis_index("c")
    s = lax.axis_index("s")
    wid = s * 2 + c

    zeros16 = jnp.zeros((16,), jnp.float32)
    ones16 = jnp.ones((16,), jnp.float32)

    def fill(i, _):
        for k in range(D // 16):
            ones_v[i, pl.ds(k * 16, 16)] = ones16
            zero_v[i, pl.ds(k * 16, 16)] = zeros16
        return 0
    lax.fori_loop(0, C, fill, 0)

    row0 = s * ROWS_PER_TILE
    for j in range(ROWS_PER_TILE // C):
        pltpu.sync_copy(zero_v, deg_sh.at[pl.ds(row0 + j * C, C)])

    plsc.subcore_barrier()

    base_edge = wid * CH_ALLOC * C
    bufs = ((dst_v0, sdst_v0, isem0, ssem0), (dst_v1, sdst_v1, isem1, ssem1))

    def start_idx(ci, b):
        dstv, _, isem, _ = bufs[b]
        pltpu.async_copy(dst_hbm.at[pl.ds(base_edge + ci * C, C)], dstv, isem)

    def wait_idx(b):
        dstv, _, isem, _ = bufs[b]
        pltpu.make_async_copy(dst_hbm.at[pl.ds(0, C)], dstv, isem).wait()

    def start_scatter(b):
        _, sdstv, _, ssem = bufs[b]
        pltpu.async_copy(ones_v, deg_sh.at[sdstv], ssem, add=True)

    def wait_scatter(b):
        _, _, _, ssem = bufs[b]
        pltpu.make_async_copy(deg_out.at[0, pl.ds(0, C)], ones_v, ssem).wait()

    def iter_body(ci, b, first):
        dstv, sdstv, _, _ = bufs[b]
        wait_idx(b)
        if not first:
            wait_scatter(1 - b)
        for k in range(C // 16):
            sdstv[pl.ds(k * 16, 16)] = dstv[pl.ds(k * 16, 16)]
        start_scatter(b)
        start_idx(ci + 2, b)

    start_idx(0, 0)
    start_idx(1, 1)
    iter_body(0, 0, first=True)
    iter_body(1, 1, first=False)

    def pair_body(g, _):
        for b in (0, 1):
            iter_body(2 * g + b, b, first=False)
        return 0

    lax.fori_loop(1, CHUNKS // 2, pair_body, 0)

    wait_idx(0)
    wait_idx(1)
    wait_scatter(1)

    plsc.subcore_barrier()

    pltpu.sync_copy(deg_sh.at[pl.ds(row0, ROWS_PER_TILE)],
                    deg_out.at[c, pl.ds(row0, ROWS_PER_TILE)])


_deg_pass = functools.partial(
    pl.kernel,
    out_type=jax.ShapeDtypeStruct((2, N_PAD, D), jnp.float32),
    mesh=_mesh,
    scratch_types=[
        pltpu.VMEM((C,), jnp.int32),
        pltpu.VMEM((C,), jnp.int32),
        pltpu.VMEM((C,), jnp.int32),
        pltpu.VMEM((C,), jnp.int32),
        pltpu.VMEM((C, D), jnp.float32),
        pltpu.VMEM((C, D), jnp.float32),
        pltpu.VMEM_SHARED((N_PAD, D), jnp.float32),
        pltpu.SemaphoreType.DMA,
        pltpu.SemaphoreType.DMA,
        pltpu.SemaphoreType.DMA,
        pltpu.SemaphoreType.DMA,
    ],
)(_deg_body)


def _edge_pass_body(x_hbm, p_hbm, st_hbm,
                    agg_out,
                    st_v0, st_v1, sdst_v0, sdst_v1,
                    h_v0, h_v1, p_v0, p_v1,
                    agg_sh, p_sh,
                    isem0, isem1, hsem0, hsem1, psem0, psem1, ssem0, ssem1):
    c = lax.axis_index("c")
    s = lax.axis_index("s")
    wid = s * 2 + c

    zeros16 = jnp.zeros((16,), jnp.float32)

    def zrow(i, _):
        for k in range(D // 16):
            h_v0[i, pl.ds(k * 16, 16)] = zeros16
        return 0
    lax.fori_loop(0, C, zrow, 0)

    @pl.when(s == 0)
    def _():
        pltpu.sync_copy(p_hbm, p_sh)

    row0 = s * ROWS_PER_TILE
    for j in range(ROWS_PER_TILE // C):
        pltpu.sync_copy(h_v0, agg_sh.at[pl.ds(row0 + j * C, C)])

    plsc.subcore_barrier()

    base_st = wid * CH_ALLOC * 3 * C
    bufs = ((st_v0, sdst_v0, h_v0, p_v0, isem0, hsem0, psem0, ssem0),
            (st_v1, sdst_v1, h_v1, p_v1, isem1, hsem1, psem1, ssem1))

    def start_idx(ci, b):
        stv, _, _, _, isem, _, _, _ = bufs[b]
        pltpu.async_copy(st_hbm.at[pl.ds(base_st + ci * 3 * C, 3 * C)],
                         stv, isem)

    def wait_idx(b):
        stv, _, _, _, isem, _, _, _ = bufs[b]
        pltpu.make_async_copy(st_hbm.at[pl.ds(0, 3 * C)], stv, isem).wait()

    def start_gather(b):
        stv, _, hv, pv, _, hsem, psem, _ = bufs[b]
        pltpu.async_copy(x_hbm.at[stv.at[pl.ds(0, C)]], hv, hsem)
        pltpu.async_copy(p_sh.at[stv.at[pl.ds(C, C)]], pv, psem)

    def wait_gather(b):
        _, _, hv, pv, _, hsem, psem, _ = bufs[b]
        pltpu.make_async_copy(x_hbm.at[pl.ds(0, C)], hv, hsem).wait()
        pltpu.make_async_copy(x_hbm.at[pl.ds(0, C)], pv, psem).wait()

    def start_scatter(b):
        _, sdstv, hv, _, _, _, _, ssem = bufs[b]
        pltpu.async_copy(hv, agg_sh.at[sdstv], ssem, add=True)

    def wait_scatter(b):
        _, _, hv, _, _, _, _, ssem = bufs[b]
        pltpu.make_async_copy(x_hbm.at[pl.ds(0, C)], hv, ssem).wait()

    lanes = lax.iota(jnp.int32, 16)
    _gdn = lax.GatherDimensionNumbers(
        offset_dims=(), collapsed_slice_dims=(0,), start_index_map=(0,))

    def _shuf(v, idx):
        return lax.gather(v, idx[:, None], _gdn, (1,),
                          mode=lax.GatherScatterMode.PROMISE_IN_BOUNDS)

    def compute_and_scatter(b):
        stv, sdstv, hv, pv, _, _, _, _ = bufs[b]

        @plsc.parallel_loop(0, C, step=1, unroll=2)
        def _edges(e):
            acc = zeros16
            for k in range(D // 16):
                acc = acc + hv[e, pl.ds(k * 16, 16)] * pv[e, pl.ds(k * 16, 16)]
            for sh in (8, 4, 2, 1):
                acc = acc + _shuf(acc, lanes ^ sh)
            for k in range(D // 16):
                hv[e, pl.ds(k * 16, 16)] = (
                    hv[e, pl.ds(k * 16, 16)] - acc * pv[e, pl.ds(k * 16, 16)]
                )

        for k in range(C // 16):
            sdstv[pl.ds(k * 16, 16)] = stv[pl.ds(2 * C + k * 16, 16)]
        start_scatter(b)

    def iter_body(ci, b, first):
        wait_idx(1 - b)
        if not first:
            wait_scatter(1 - b)
        start_gather(1 - b)
        wait_gather(b)
        compute_and_scatter(b)
        start_idx(ci + 2, b)

    start_idx(0, 0)
    wait_idx(0)
    start_gather(0)
    start_idx(1, 1)
    iter_body(0, 0, first=True)
    iter_body(1, 1, first=False)

    def pair_body(g, _):
        for b in (0, 1):
            iter_body(2 * g + b, b, first=False)
        return 0

    lax.fori_loop(1, CHUNKS // 2, pair_body, 0)

    wait_gather(0)
    wait_idx(1)
    wait_scatter(1)

    plsc.subcore_barrier()

    pltpu.sync_copy(agg_sh.at[pl.ds(row0, ROWS_PER_TILE)],
                    agg_out.at[c, pl.ds(row0, ROWS_PER_TILE)])


_edge_pass = functools.partial(
    pl.kernel,
    out_type=jax.ShapeDtypeStruct((2, N_PAD, D), jnp.float32),
    mesh=_mesh,
    scratch_types=[
        pltpu.VMEM((3 * C,), jnp.int32),
        pltpu.VMEM((3 * C,), jnp.int32),
        pltpu.VMEM((C,), jnp.int32),
        pltpu.VMEM((C,), jnp.int32),
        pltpu.VMEM((C, D), jnp.float32),
        pltpu.VMEM((C, D), jnp.float32),
        pltpu.VMEM((C, D), jnp.float32),
        pltpu.VMEM((C, D), jnp.float32),
        pltpu.VMEM_SHARED((N_PAD, D), jnp.float32),
        pltpu.VMEM_SHARED((R_PAD, D), jnp.float32),
        pltpu.SemaphoreType.DMA,
        pltpu.SemaphoreType.DMA,
        pltpu.SemaphoreType.DMA,
        pltpu.SemaphoreType.DMA,
        pltpu.SemaphoreType.DMA,
        pltpu.SemaphoreType.DMA,
        pltpu.SemaphoreType.DMA,
        pltpu.SemaphoreType.DMA,
    ],
)(_edge_pass_body)


def _prep_body(r_ref, o_ref):
    r = r_ref[...]
    norm = jnp.sqrt(jnp.sum(r * r, axis=1, keepdims=True))
    o_ref[...] = r * (np.float32(np.sqrt(2.0)) / (norm + 1e-8))


_prep = pl.pallas_call(
    _prep_body,
    out_shape=jax.ShapeDtypeStruct((R_PAD, D), jnp.float32),
)


def _layer_body(a_ref, d_ref, w_ref, o_ref):
    deg = jnp.maximum(d_ref[0, :, 0:1] + d_ref[1, :, 0:1], 1.0)
    x = (a_ref[0] + a_ref[1]) / deg
    o_ref[...] = jnp.maximum(
        jnp.dot(x, w_ref[...], preferred_element_type=jnp.float32), 0.0)


_layer = pl.pallas_call(
    _layer_body,
    out_shape=jax.ShapeDtypeStruct((N_PAD, D), jnp.float32),
)


def _final_body(a_ref, d_ref, w2_ref, l1w_ref, l1b_ref, g_ref, b_ref,
                l2w_ref, l2b_ref, o_ref):
    deg = jnp.maximum(d_ref[0, :, 0:1] + d_ref[1, :, 0:1], 1.0)
    x = (a_ref[0] + a_ref[1]) / deg
    x = jnp.maximum(
        jnp.dot(x, w2_ref[...], preferred_element_type=jnp.float32), 0.0)
    h = jnp.dot(x, l1w_ref[...], preferred_element_type=jnp.float32) + l1b_ref[...]
    mask = (lax.broadcasted_iota(jnp.int32, (N_PAD, 1), 0) < N).astype(jnp.float32)
    cnt = np.float32(N)
    mean = jnp.sum(h * mask, axis=0, keepdims=True) / cnt
    var = jnp.sum((h - mean) ** 2 * mask, axis=0, keepdims=True) / cnt
    h = (h - mean) / jnp.sqrt(var + 1e-5) * g_ref[...] + b_ref[...]
    h = jnp.maximum(h, 0.0)
    o_ref[...] = jnp.dot(h, l2w_ref[...], preferred_element_type=jnp.float32) + l2b_ref[...]


_final = pl.pallas_call(
    _final_body,
    out_shape=jax.ShapeDtypeStruct((N_PAD, D), jnp.float32),
)


def kernel(edge_index, edge_type, initial_features, relation_embeddings,
           W1, W2, lin1_w, lin1_b, bn_gamma, bn_beta, lin2_w, lin2_b):
    pad = E_PAD - E
    src = jnp.concatenate(
        [edge_index[0].astype(jnp.int32), jnp.zeros((pad,), jnp.int32)])
    dst = jnp.concatenate(
        [edge_index[1].astype(jnp.int32), jnp.full((pad,), N, jnp.int32)])
    typ = jnp.concatenate(
        [edge_type.astype(jnp.int32), jnp.zeros((pad,), jnp.int32)])

    src_r = jnp.pad(src.reshape(NUM_TILES, CHUNKS, C), ((0, 0), (0, 2), (0, 0)))
    typ_r = jnp.pad(typ.reshape(NUM_TILES, CHUNKS, C), ((0, 0), (0, 2), (0, 0)))
    dst_r = jnp.pad(dst.reshape(NUM_TILES, CHUNKS, C), ((0, 0), (0, 2), (0, 0)),
                    constant_values=N)
    st = jnp.stack([src_r, typ_r, dst_r], axis=2).reshape(-1)
    dstc = dst_r.reshape(-1)

    x0 = jnp.pad(initial_features, ((0, N_PAD - N), (0, 0)))
    relp = jnp.pad(relation_embeddings, ((0, R_PAD - R), (0, 0)))

    p = _prep(relp)
    deg = _deg_pass(dstc)

    agg1 = _edge_pass(x0, p, st)
    x1 = _layer(agg1, deg, W1)
    agg2 = _edge_pass(x1, p, st)
    out = _final(agg2, deg, W2, lin1_w, lin1_b.reshape(1, D),
                 bn_gamma.reshape(1, D), bn_beta.reshape(1, D),
                 lin2_w, lin2_b.reshape(1, D))
    return out[:N]


# speedup vs baseline: 1.4484x; 1.3717x over previous
import functools

import jax
import jax.numpy as jnp
import numpy as np
from jax import lax
from jax.experimental import pallas as pl
from jax.experimental.pallas import tpu as pltpu
from jax.experimental.pallas import tpu_sc as plsc

N = 10000
E = 320000
R = 1001
D = 128

NUM_TILES = 32
C = 48
CHUNKS = 4 * (-(-E // (4 * C * NUM_TILES)))
CH_ALLOC = CHUNKS + 2
EPT = CHUNKS * C
E_PAD = EPT * NUM_TILES
N_PAD = 10240
R_PAD = 1008
ROWS_PER_TILE = N_PAD // 16

_mesh = plsc.VectorSubcoreMesh(core_axis_name="c", subcore_axis_name="s")


def _deg_body(dst_hbm, deg_out,
              dst_v0, dst_v1, sdst_v0, sdst_v1, ones_v, zero_v, deg_sh,
              isem0, isem1, ssem0, ssem1):
    c = lax.axis_index("c")
    s = lax.axis_index("s")
    wid = s * 2 + c

    zeros16 = jnp.zeros((16,), jnp.float32)
    ones16 = jnp.ones((16,), jnp.float32)

    def fill(i, _):
        for k in range(D // 16):
            ones_v[i, pl.ds(k * 16, 16)] = ones16
            zero_v[i, pl.ds(k * 16, 16)] = zeros16
        return 0
    lax.fori_loop(0, C, fill, 0)

    row0 = s * ROWS_PER_TILE
    for j in range(ROWS_PER_TILE // 40):
        pltpu.sync_copy(zero_v.at[pl.ds(0, 40)],
                        deg_sh.at[pl.ds(row0 + j * 40, 40)])

    plsc.subcore_barrier()

    base_edge = wid * CH_ALLOC * C
    bufs = ((dst_v0, sdst_v0, isem0, ssem0), (dst_v1, sdst_v1, isem1, ssem1))

    def start_idx(ci, b):
        dstv, _, isem, _ = bufs[b]
        pltpu.async_copy(dst_hbm.at[pl.ds(base_edge + ci * C, C)], dstv, isem)

    def wait_idx(b):
        dstv, _, isem, _ = bufs[b]
        pltpu.make_async_copy(dst_hbm.at[pl.ds(0, C)], dstv, isem).wait()

    def start_scatter(b):
        _, sdstv, _, ssem = bufs[b]
        pltpu.async_copy(ones_v, deg_sh.at[sdstv], ssem, add=True)

    def wait_scatter(b):
        _, _, _, ssem = bufs[b]
        pltpu.make_async_copy(deg_out.at[0, pl.ds(0, C)], ones_v, ssem).wait()

    def iter_body(ci, b, first):
        dstv, sdstv, _, _ = bufs[b]
        wait_idx(b)
        if not first:
            wait_scatter(1 - b)
        for k in range(C // 16):
            sdstv[pl.ds(k * 16, 16)] = dstv[pl.ds(k * 16, 16)]
        start_scatter(b)
        start_idx(ci + 2, b)

    start_idx(0, 0)
    start_idx(1, 1)
    iter_body(0, 0, first=True)
    iter_body(1, 1, first=False)

    def pair_body(g, _):
        for b in (0, 1):
            iter_body(2 * g + b, b, first=False)
        return 0

    lax.fori_loop(1, CHUNKS // 2, pair_body, 0)

    wait_idx(0)
    wait_idx(1)
    wait_scatter(1)

    plsc.subcore_barrier()

    pltpu.sync_copy(deg_sh.at[pl.ds(row0, ROWS_PER_TILE)],
                    deg_out.at[c, pl.ds(row0, ROWS_PER_TILE)])


_deg_pass = functools.partial(
    pl.kernel,
    out_type=jax.ShapeDtypeStruct((2, N_PAD, D), jnp.float32),
    mesh=_mesh,
    scratch_types=[
        pltpu.VMEM((C,), jnp.int32),
        pltpu.VMEM((C,), jnp.int32),
        pltpu.VMEM((C,), jnp.int32),
        pltpu.VMEM((C,), jnp.int32),
        pltpu.VMEM((C, D), jnp.float32),
        pltpu.VMEM((C, D), jnp.float32),
        pltpu.VMEM_SHARED((N_PAD, D), jnp.float32),
        pltpu.SemaphoreType.DMA,
        pltpu.SemaphoreType.DMA,
        pltpu.SemaphoreType.DMA,
        pltpu.SemaphoreType.DMA,
    ],
)(_deg_body)


def _edge_pass_body(x_hbm, p_hbm, st_hbm,
                    agg_out,
                    st_v0, st_v1, sdst_v0, sdst_v1,
                    h_v0, h_v1, p_v0, p_v1, m_v0, m_v1,
                    agg_sh, p_sh,
                    isem0, isem1, hsem0, hsem1, psem0, psem1, ssem0, ssem1):
    c = lax.axis_index("c")
    s = lax.axis_index("s")
    wid = s * 2 + c

    zeros16 = jnp.zeros((16,), jnp.float32)

    def zrow(i, _):
        for k in range(D // 16):
            m_v0[i, pl.ds(k * 16, 16)] = zeros16
        return 0
    lax.fori_loop(0, C, zrow, 0)

    @pl.when(s == 0)
    def _():
        pltpu.sync_copy(p_hbm, p_sh)

    row0 = s * ROWS_PER_TILE
    for j in range(ROWS_PER_TILE // 40):
        pltpu.sync_copy(m_v0.at[pl.ds(0, 40)],
                        agg_sh.at[pl.ds(row0 + j * 40, 40)])

    plsc.subcore_barrier()

    base_st = wid * CH_ALLOC * 3 * C
    bufs = ((st_v0, sdst_v0, h_v0, p_v0, m_v0, isem0, hsem0, psem0, ssem0),
            (st_v1, sdst_v1, h_v1, p_v1, m_v1, isem1, hsem1, psem1, ssem1))

    def start_idx(ci, b):
        stv, _, _, _, _, isem, _, _, _ = bufs[b]
        pltpu.async_copy(st_hbm.at[pl.ds(base_st + ci * 3 * C, 3 * C)],
                         stv, isem)

    def wait_idx(b):
        stv, _, _, _, _, isem, _, _, _ = bufs[b]
        pltpu.make_async_copy(st_hbm.at[pl.ds(0, 3 * C)], stv, isem).wait()

    def start_gather(b):
        stv, _, hv, pv, _, _, hsem, psem, _ = bufs[b]
        pltpu.async_copy(x_hbm.at[stv.at[pl.ds(0, C)]], hv, hsem)
        pltpu.async_copy(p_sh.at[stv.at[pl.ds(C, C)]], pv, psem)

    def wait_gather(b):
        _, _, hv, pv, _, _, hsem, psem, _ = bufs[b]
        pltpu.make_async_copy(x_hbm.at[pl.ds(0, C)], hv, hsem).wait()
        pltpu.make_async_copy(x_hbm.at[pl.ds(0, C)], pv, psem).wait()

    def start_scatter(b):
        _, sdstv, _, _, mv, _, _, _, ssem = bufs[b]
        pltpu.async_copy(mv, agg_sh.at[sdstv], ssem, add=True)

    def wait_scatter(b):
        _, _, _, _, mv, _, _, _, ssem = bufs[b]
        pltpu.make_async_copy(x_hbm.at[pl.ds(0, C)], mv, ssem).wait()

    lanes = lax.iota(jnp.int32, 16)
    _gdn = lax.GatherDimensionNumbers(
        offset_dims=(), collapsed_slice_dims=(0,), start_index_map=(0,))

    def _shuf(v, idx):
        return lax.gather(v, idx[:, None], _gdn, (1,),
                          mode=lax.GatherScatterMode.PROMISE_IN_BOUNDS)

    def compute_and_scatter(b):
        stv, sdstv, hv, pv, mv, _, _, _, _ = bufs[b]

        @plsc.parallel_loop(0, C, step=1, unroll=2)
        def _edges(e):
            acc = zeros16
            for k in range(D // 16):
                acc = acc + hv[e, pl.ds(k * 16, 16)] * pv[e, pl.ds(k * 16, 16)]
            for sh in (8, 4, 2, 1):
                acc = acc + _shuf(acc, lanes ^ sh)
            for k in range(D // 16):
                mv[e, pl.ds(k * 16, 16)] = (
                    hv[e, pl.ds(k * 16, 16)] - acc * pv[e, pl.ds(k * 16, 16)]
                )

        for k in range(C // 16):
            sdstv[pl.ds(k * 16, 16)] = stv[pl.ds(2 * C + k * 16, 16)]
        start_scatter(b)

    def iter_body(ci, b, first):
        wait_idx(1 - b)
        start_gather(1 - b)
        if not first:
            wait_scatter(b)
        wait_gather(b)
        compute_and_scatter(b)
        start_idx(ci + 2, b)

    start_idx(0, 0)
    wait_idx(0)
    start_gather(0)
    start_idx(1, 1)
    iter_body(0, 0, first=True)
    iter_body(1, 1, first=True)

    def pair_body(g, _):
        for b in (0, 1):
            iter_body(2 * g + b, b, first=False)
        return 0

    lax.fori_loop(1, CHUNKS // 2, pair_body, 0)

    wait_gather(0)
    wait_idx(1)
    wait_scatter(0)
    wait_scatter(1)

    plsc.subcore_barrier()

    pltpu.sync_copy(agg_sh.at[pl.ds(row0, ROWS_PER_TILE)],
                    agg_out.at[c, pl.ds(row0, ROWS_PER_TILE)])


_edge_pass = functools.partial(
    pl.kernel,
    out_type=jax.ShapeDtypeStruct((2, N_PAD, D), jnp.float32),
    mesh=_mesh,
    scratch_types=[
        pltpu.VMEM((3 * C,), jnp.int32),
        pltpu.VMEM((3 * C,), jnp.int32),
        pltpu.VMEM((C,), jnp.int32),
        pltpu.VMEM((C,), jnp.int32),
        pltpu.VMEM((C, D), jnp.float32),
        pltpu.VMEM((C, D), jnp.float32),
        pltpu.VMEM((C, D), jnp.float32),
        pltpu.VMEM((C, D), jnp.float32),
        pltpu.VMEM((C, D), jnp.float32),
        pltpu.VMEM((C, D), jnp.float32),
        pltpu.VMEM_SHARED((N_PAD, D), jnp.float32),
        pltpu.VMEM_SHARED((R_PAD, D), jnp.float32),
        pltpu.SemaphoreType.DMA,
        pltpu.SemaphoreType.DMA,
        pltpu.SemaphoreType.DMA,
        pltpu.SemaphoreType.DMA,
        pltpu.SemaphoreType.DMA,
        pltpu.SemaphoreType.DMA,
        pltpu.SemaphoreType.DMA,
        pltpu.SemaphoreType.DMA,
    ],
)(_edge_pass_body)


def _prep_body(r_ref, o_ref):
    r = r_ref[...]
    norm = jnp.sqrt(jnp.sum(r * r, axis=1, keepdims=True))
    o_ref[...] = r * (np.float32(np.sqrt(2.0)) / (norm + 1e-8))


_prep = pl.pallas_call(
    _prep_body,
    out_shape=jax.ShapeDtypeStruct((R_PAD, D), jnp.float32),
)


def _layer_body(a_ref, d_ref, w_ref, o_ref):
    deg = jnp.maximum(d_ref[0, :, 0:1] + d_ref[1, :, 0:1], 1.0)
    x = (a_ref[0] + a_ref[1]) / deg
    o_ref[...] = jnp.maximum(
        jnp.dot(x, w_ref[...], preferred_element_type=jnp.float32), 0.0)


_layer = pl.pallas_call(
    _layer_body,
    out_shape=jax.ShapeDtypeStruct((N_PAD, D), jnp.float32),
)


def _final_body(a_ref, d_ref, w2_ref, l1w_ref, l1b_ref, g_ref, b_ref,
                l2w_ref, l2b_ref, o_ref):
    deg = jnp.maximum(d_ref[0, :, 0:1] + d_ref[1, :, 0:1], 1.0)
    x = (a_ref[0] + a_ref[1]) / deg
    x = jnp.maximum(
        jnp.dot(x, w2_ref[...], preferred_element_type=jnp.float32), 0.0)
    h = jnp.dot(x, l1w_ref[...], preferred_element_type=jnp.float32) + l1b_ref[...]
    mask = (lax.broadcasted_iota(jnp.int32, (N_PAD, 1), 0) < N).astype(jnp.float32)
    cnt = np.float32(N)
    mean = jnp.sum(h * mask, axis=0, keepdims=True) / cnt
    var = jnp.sum((h - mean) ** 2 * mask, axis=0, keepdims=True) / cnt
    h = (h - mean) / jnp.sqrt(var + 1e-5) * g_ref[...] + b_ref[...]
    h = jnp.maximum(h, 0.0)
    o_ref[...] = jnp.dot(h, l2w_ref[...], preferred_element_type=jnp.float32) + l2b_ref[...]


_final = pl.pallas_call(
    _final_body,
    out_shape=jax.ShapeDtypeStruct((N_PAD, D), jnp.float32),
)


def kernel(edge_index, edge_type, initial_features, relation_embeddings,
           W1, W2, lin1_w, lin1_b, bn_gamma, bn_beta, lin2_w, lin2_b):
    pad = E_PAD - E
    pad_ids = jnp.arange(pad, dtype=jnp.int32)
    src = jnp.concatenate(
        [edge_index[0].astype(jnp.int32), pad_ids % N])
    dst = jnp.concatenate(
        [edge_index[1].astype(jnp.int32), N + pad_ids % (N_PAD - N)])
    typ = jnp.concatenate(
        [edge_type.astype(jnp.int32), jnp.zeros((pad,), jnp.int32)])

    src_r = jnp.pad(src.reshape(NUM_TILES, CHUNKS, C), ((0, 0), (0, 2), (0, 0)))
    typ_r = jnp.pad(typ.reshape(NUM_TILES, CHUNKS, C), ((0, 0), (0, 2), (0, 0)))
    dst_r = jnp.pad(dst.reshape(NUM_TILES, CHUNKS, C), ((0, 0), (0, 2), (0, 0)),
                    constant_values=N)
    st = jnp.stack([src_r, typ_r, dst_r], axis=2).reshape(-1)
    dstc = dst_r.reshape(-1)

    x0 = jnp.pad(initial_features, ((0, N_PAD - N), (0, 0)))
    relp = jnp.pad(relation_embeddings, ((0, R_PAD - R), (0, 0)))

    p = _prep(relp)
    deg = _deg_pass(dstc)

    agg1 = _edge_pass(x0, p, st)
    x1 = _layer(agg1, deg, W1)
    agg2 = _edge_pass(x1, p, st)
    out = _final(agg2, deg, W2, lin1_w, lin1_b.reshape(1, D),
                 bn_gamma.reshape(1, D), bn_beta.reshape(1, D),
                 lin2_w, lin2_b.reshape(1, D))
    return out[:N]


# speedup vs baseline: 1.4549x; 1.0045x over previous
import functools

import jax
import jax.numpy as jnp
import numpy as np
from jax import lax
from jax.experimental import pallas as pl
from jax.experimental.pallas import tpu as pltpu
from jax.experimental.pallas import tpu_sc as plsc

N = 10000
E = 320000
R = 1001
D = 128

NUM_TILES = 32
C = 48
CHUNKS = 4 * (-(-E // (4 * C * NUM_TILES)))
CH_ALLOC = CHUNKS + 2
EPT = CHUNKS * C
E_PAD = EPT * NUM_TILES
N_PAD = 10240
R_PAD = 1008
ROWS_PER_TILE = N_PAD // 16

_mesh = plsc.VectorSubcoreMesh(core_axis_name="c", subcore_axis_name="s")


def _deg_body(dst_hbm, deg_out,
              dst_v0, dst_v1, sdst_v0, sdst_v1, ones_v, zero_v, deg_sh,
              isem0, isem1, ssem0, ssem1):
    c = lax.axis_index("c")
    s = lax.axis_index("s")
    wid = s * 2 + c

    zeros16 = jnp.zeros((16,), jnp.float32)
    ones16 = jnp.ones((16,), jnp.float32)

    def fill(i, _):
        for k in range(D // 16):
            ones_v[i, pl.ds(k * 16, 16)] = ones16
            zero_v[i, pl.ds(k * 16, 16)] = zeros16
        return 0
    lax.fori_loop(0, C, fill, 0)

    row0 = s * ROWS_PER_TILE
    for j in range(ROWS_PER_TILE // 40):
        pltpu.sync_copy(zero_v.at[pl.ds(0, 40)],
                        deg_sh.at[pl.ds(row0 + j * 40, 40)])

    plsc.subcore_barrier()

    base_edge = wid * CH_ALLOC * C
    bufs = ((dst_v0, sdst_v0, isem0, ssem0), (dst_v1, sdst_v1, isem1, ssem1))

    def start_idx(ci, b):
        dstv, _, isem, _ = bufs[b]
        pltpu.async_copy(dst_hbm.at[pl.ds(base_edge + ci * C, C)], dstv, isem)

    def wait_idx(b):
        dstv, _, isem, _ = bufs[b]
        pltpu.make_async_copy(dst_hbm.at[pl.ds(0, C)], dstv, isem).wait()

    def start_scatter(b):
        _, sdstv, _, ssem = bufs[b]
        pltpu.async_copy(ones_v, deg_sh.at[sdstv], ssem, add=True)

    def wait_scatter(b):
        _, _, _, ssem = bufs[b]
        pltpu.make_async_copy(deg_out.at[0, pl.ds(0, C)], ones_v, ssem).wait()

    def iter_body(ci, b, first):
        dstv, sdstv, _, _ = bufs[b]
        wait_idx(b)
        if not first:
            wait_scatter(1 - b)
        for k in range(C // 16):
            sdstv[pl.ds(k * 16, 16)] = dstv[pl.ds(k * 16, 16)]
        start_scatter(b)
        start_idx(ci + 2, b)

    start_idx(0, 0)
    start_idx(1, 1)
    iter_body(0, 0, first=True)
    iter_body(1, 1, first=False)

    def pair_body(g, _):
        for b in (0, 1):
            iter_body(2 * g + b, b, first=False)
        return 0

    lax.fori_loop(1, CHUNKS // 2, pair_body, 0)

    wait_idx(0)
    wait_idx(1)
    wait_scatter(1)

    plsc.subcore_barrier()

    pltpu.sync_copy(deg_sh.at[pl.ds(row0, ROWS_PER_TILE)],
                    deg_out.at[c, pl.ds(row0, ROWS_PER_TILE)])


_deg_pass = functools.partial(
    pl.kernel,
    out_type=jax.ShapeDtypeStruct((2, N_PAD, D), jnp.float32),
    mesh=_mesh,
    scratch_types=[
        pltpu.VMEM((C,), jnp.int32),
        pltpu.VMEM((C,), jnp.int32),
        pltpu.VMEM((C,), jnp.int32),
        pltpu.VMEM((C,), jnp.int32),
        pltpu.VMEM((C, D), jnp.float32),
        pltpu.VMEM((C, D), jnp.float32),
        pltpu.VMEM_SHARED((N_PAD, D), jnp.float32),
        pltpu.SemaphoreType.DMA,
        pltpu.SemaphoreType.DMA,
        pltpu.SemaphoreType.DMA,
        pltpu.SemaphoreType.DMA,
    ],
)(_deg_body)


def _edge_pass_body(x_hbm, p_hbm, st_hbm,
                    agg_out,
                    st_v0, st_v1, sdst_v0, sdst_v1,
                    h_v0, h_v1, p_v0, p_v1, m_v0, m_v1,
                    agg_sh, p_sh,
                    isem0, isem1, hsem0, hsem1, psem0, psem1, ssem0, ssem1):
    c = lax.axis_index("c")
    s = lax.axis_index("s")
    wid = s * 2 + c

    zeros16 = jnp.zeros((16,), jnp.float32)

    def zrow(i, _):
        for k in range(D // 16):
            m_v0[i, pl.ds(k * 16, 16)] = zeros16
        return 0
    lax.fori_loop(0, C, zrow, 0)

    @pl.when(s == 0)
    def _():
        pltpu.sync_copy(p_hbm, p_sh)

    row0 = s * ROWS_PER_TILE
    for j in range(ROWS_PER_TILE // 40):
        pltpu.sync_copy(m_v0.at[pl.ds(0, 40)],
                        agg_sh.at[pl.ds(row0 + j * 40, 40)])

    plsc.subcore_barrier()

    base_st = wid * CH_ALLOC * 3 * C
    bufs = ((st_v0, sdst_v0, h_v0, p_v0, m_v0, isem0, hsem0, psem0, ssem0),
            (st_v1, sdst_v1, h_v1, p_v1, m_v1, isem1, hsem1, psem1, ssem1))

    def start_idx(ci, b):
        stv, _, _, _, _, isem, _, _, _ = bufs[b]
        pltpu.async_copy(st_hbm.at[pl.ds(base_st + ci * 3 * C, 3 * C)],
                         stv, isem)

    def wait_idx(b):
        stv, _, _, _, _, isem, _, _, _ = bufs[b]
        pltpu.make_async_copy(st_hbm.at[pl.ds(0, 3 * C)], stv, isem).wait()

    def start_gather(b):
        stv, _, hv, pv, _, _, hsem, psem, _ = bufs[b]
        pltpu.async_copy(x_hbm.at[stv.at[pl.ds(0, C)]], hv, hsem)
        pltpu.async_copy(p_sh.at[stv.at[pl.ds(C, C)]], pv, psem)

    def wait_gather(b):
        _, _, hv, pv, _, _, hsem, psem, _ = bufs[b]
        pltpu.make_async_copy(x_hbm.at[pl.ds(0, C)], hv, hsem).wait()
        pltpu.make_async_copy(x_hbm.at[pl.ds(0, C)], pv, psem).wait()

    def start_scatter(b):
        _, sdstv, _, _, mv, _, _, _, ssem = bufs[b]
        pltpu.async_copy(mv, agg_sh.at[sdstv], ssem, add=True)

    def wait_scatter(b):
        _, _, _, _, mv, _, _, _, ssem = bufs[b]
        pltpu.make_async_copy(x_hbm.at[pl.ds(0, C)], mv, ssem).wait()

    lanes = lax.iota(jnp.int32, 16)
    _gdn = lax.GatherDimensionNumbers(
        offset_dims=(), collapsed_slice_dims=(0,), start_index_map=(0,))

    def _shuf(v, idx):
        return lax.gather(v, idx[:, None], _gdn, (1,),
                          mode=lax.GatherScatterMode.PROMISE_IN_BOUNDS)

    def compute_and_scatter(b):
        stv, sdstv, hv, pv, mv, _, _, _, _ = bufs[b]

        @plsc.parallel_loop(0, C, step=1, unroll=4)
        def _edges(e):
            acc = zeros16
            for k in range(D // 16):
                acc = acc + hv[e, pl.ds(k * 16, 16)] * pv[e, pl.ds(k * 16, 16)]
            for sh in (8, 4, 2, 1):
                acc = acc + _shuf(acc, lanes ^ sh)
            for k in range(D // 16):
                mv[e, pl.ds(k * 16, 16)] = (
                    hv[e, pl.ds(k * 16, 16)] - acc * pv[e, pl.ds(k * 16, 16)]
                )

        for k in range(C // 16):
            sdstv[pl.ds(k * 16, 16)] = stv[pl.ds(2 * C + k * 16, 16)]
        start_scatter(b)

    def iter_body(ci, b, first):
        wait_idx(1 - b)
        start_gather(1 - b)
        if not first:
            wait_scatter(b)
        wait_gather(b)
        compute_and_scatter(b)
        start_idx(ci + 2, b)

    start_idx(0, 0)
    wait_idx(0)
    start_gather(0)
    start_idx(1, 1)
    iter_body(0, 0, first=True)
    iter_body(1, 1, first=True)

    def pair_body(g, _):
        for b in (0, 1):
            iter_body(2 * g + b, b, first=False)
        return 0

    lax.fori_loop(1, CHUNKS // 2, pair_body, 0)

    wait_gather(0)
    wait_idx(1)
    wait_scatter(0)
    wait_scatter(1)

    plsc.subcore_barrier()

    pltpu.sync_copy(agg_sh.at[pl.ds(row0, ROWS_PER_TILE)],
                    agg_out.at[c, pl.ds(row0, ROWS_PER_TILE)])


_edge_pass = functools.partial(
    pl.kernel,
    out_type=jax.ShapeDtypeStruct((2, N_PAD, D), jnp.float32),
    mesh=_mesh,
    scratch_types=[
        pltpu.VMEM((3 * C,), jnp.int32),
        pltpu.VMEM((3 * C,), jnp.int32),
        pltpu.VMEM((C,), jnp.int32),
        pltpu.VMEM((C,), jnp.int32),
        pltpu.VMEM((C, D), jnp.float32),
        pltpu.VMEM((C, D), jnp.float32),
        pltpu.VMEM((C, D), jnp.float32),
        pltpu.VMEM((C, D), jnp.float32),
        pltpu.VMEM((C, D), jnp.float32),
        pltpu.VMEM((C, D), jnp.float32),
        pltpu.VMEM_SHARED((N_PAD, D), jnp.float32),
        pltpu.VMEM_SHARED((R_PAD, D), jnp.float32),
        pltpu.SemaphoreType.DMA,
        pltpu.SemaphoreType.DMA,
        pltpu.SemaphoreType.DMA,
        pltpu.SemaphoreType.DMA,
        pltpu.SemaphoreType.DMA,
        pltpu.SemaphoreType.DMA,
        pltpu.SemaphoreType.DMA,
        pltpu.SemaphoreType.DMA,
    ],
)(_edge_pass_body)


def _prep_body(r_ref, o_ref):
    r = r_ref[...]
    norm = jnp.sqrt(jnp.sum(r * r, axis=1, keepdims=True))
    o_ref[...] = r * (np.float32(np.sqrt(2.0)) / (norm + 1e-8))


_prep = pl.pallas_call(
    _prep_body,
    out_shape=jax.ShapeDtypeStruct((R_PAD, D), jnp.float32),
)


def _layer_body(a_ref, d_ref, w_ref, o_ref):
    deg = jnp.maximum(d_ref[0, :, 0:1] + d_ref[1, :, 0:1], 1.0)
    x = (a_ref[0] + a_ref[1]) / deg
    o_ref[...] = jnp.maximum(
        jnp.dot(x, w_ref[...], preferred_element_type=jnp.float32), 0.0)


_layer = pl.pallas_call(
    _layer_body,
    out_shape=jax.ShapeDtypeStruct((N_PAD, D), jnp.float32),
)


def _final_body(a_ref, d_ref, w2_ref, l1w_ref, l1b_ref, g_ref, b_ref,
                l2w_ref, l2b_ref, o_ref):
    deg = jnp.maximum(d_ref[0, :, 0:1] + d_ref[1, :, 0:1], 1.0)
    x = (a_ref[0] + a_ref[1]) / deg
    x = jnp.maximum(
        jnp.dot(x, w2_ref[...], preferred_element_type=jnp.float32), 0.0)
    h = jnp.dot(x, l1w_ref[...], preferred_element_type=jnp.float32) + l1b_ref[...]
    mask = (lax.broadcasted_iota(jnp.int32, (N_PAD, 1), 0) < N).astype(jnp.float32)
    cnt = np.float32(N)
    mean = jnp.sum(h * mask, axis=0, keepdims=True) / cnt
    var = jnp.sum((h - mean) ** 2 * mask, axis=0, keepdims=True) / cnt
    h = (h - mean) / jnp.sqrt(var + 1e-5) * g_ref[...] + b_ref[...]
    h = jnp.maximum(h, 0.0)
    o_ref[...] = jnp.dot(h, l2w_ref[...], preferred_element_type=jnp.float32) + l2b_ref[...]


_final = pl.pallas_call(
    _final_body,
    out_shape=jax.ShapeDtypeStruct((N_PAD, D), jnp.float32),
)


def kernel(edge_index, edge_type, initial_features, relation_embeddings,
           W1, W2, lin1_w, lin1_b, bn_gamma, bn_beta, lin2_w, lin2_b):
    pad = E_PAD - E
    pad_ids = jnp.arange(pad, dtype=jnp.int32)
    src = jnp.concatenate(
        [edge_index[0].astype(jnp.int32), pad_ids % N])
    dst = jnp.concatenate(
        [edge_index[1].astype(jnp.int32), N + pad_ids % (N_PAD - N)])
    typ = jnp.concatenate(
        [edge_type.astype(jnp.int32), jnp.zeros((pad,), jnp.int32)])

    src_r = jnp.pad(src.reshape(NUM_TILES, CHUNKS, C), ((0, 0), (0, 2), (0, 0)))
    typ_r = jnp.pad(typ.reshape(NUM_TILES, CHUNKS, C), ((0, 0), (0, 2), (0, 0)))
    dst_r = jnp.pad(dst.reshape(NUM_TILES, CHUNKS, C), ((0, 0), (0, 2), (0, 0)),
                    constant_values=N)
    st = jnp.stack([src_r, typ_r, dst_r], axis=2).reshape(-1)
    dstc = dst_r.reshape(-1)

    x0 = jnp.pad(initial_features, ((0, N_PAD - N), (0, 0)))
    relp = jnp.pad(relation_embeddings, ((0, R_PAD - R), (0, 0)))

    p = _prep(relp)
    deg = _deg_pass(dstc)

    agg1 = _edge_pass(x0, p, st)
    x1 = _layer(agg1, deg, W1)
    agg2 = _edge_pass(x1, p, st)
    out = _final(agg2, deg, W2, lin1_w, lin1_b.reshape(1, D),
                 bn_gamma.reshape(1, D), bn_beta.reshape(1, D),
                 lin2_w, lin2_b.reshape(1, D))
    return out[:N]
